# baseline (speedup 1.0000x reference)
import functools

import jax
import jax.numpy as jnp
from jax import lax
from jax.experimental import pallas as pl
from jax.experimental.pallas import tpu as pltpu
from jax.experimental.pallas import tpu_sc as plsc

VOCAB = 1000000
EMBED_DIM = 64
SEQ_LEN = 200
BATCH = 4096

NUM_CORES = 2
NUM_SUBCORES = 16
NUM_WORKERS = NUM_CORES * NUM_SUBCORES
SEQ_PER_WORKER = BATCH // NUM_WORKERS
LANES = 16
VREGS_PER_ROW = EMBED_DIM // LANES
NBUF = 4

_mesh = plsc.VectorSubcoreMesh(core_axis_name="c", subcore_axis_name="s")


@functools.partial(
    pl.kernel,
    mesh=_mesh,
    out_type=jax.ShapeDtypeStruct((BATCH * SEQ_LEN, EMBED_DIM), jnp.float32),
    scratch_types=[
        pltpu.VMEM((SEQ_PER_WORKER, SEQ_LEN), jnp.int32),
        [pltpu.VMEM((SEQ_LEN, EMBED_DIM), jnp.float32) for _ in range(NBUF)],
        pltpu.VMEM((SEQ_LEN, EMBED_DIM), jnp.float32),
        [pltpu.SemaphoreType.DMA for _ in range(NBUF)],
        [pltpu.SemaphoreType.DMA for _ in range(NBUF)],
    ],
    compiler_params=pltpu.CompilerParams(use_tc_tiling_on_sc=False),
)
def _embed(ids_hbm, wt_hbm, pos_hbm, out_hbm, idx_all, rows, pos_v, gsem, ssem):
    wid = lax.axis_index("s") * NUM_CORES + lax.axis_index("c")
    seq0 = wid * SEQ_PER_WORKER
    pltpu.sync_copy(pos_hbm, pos_v)
    pltpu.sync_copy(ids_hbm.at[pl.ds(seq0, SEQ_PER_WORKER)], idx_all)

    def gather_start(c, b):
        pltpu.async_copy(wt_hbm.at[idx_all.at[c]], rows[b], gsem[b])

    def gather_wait(c, b):
        pltpu.make_async_copy(wt_hbm.at[idx_all.at[c]], rows[b], gsem[b]).wait()

    def out_slice(c):
        return out_hbm.at[pl.ds((seq0 + c) * SEQ_LEN, SEQ_LEN)]

    def store_start(c, b):
        pltpu.async_copy(rows[b], out_slice(c), ssem[b])

    def store_wait(c, b):
        pltpu.make_async_copy(rows[b], out_slice(c), ssem[b]).wait()

    def add_pos(b):
        dst = rows[b]

        def row_body(i, carry):
            for j in range(VREGS_PER_ROW):
                sl = pl.ds(j * LANES, LANES)
                dst[i, sl] = dst[i, sl] + pos_v[i, sl]
            return carry

        lax.fori_loop(0, SEQ_LEN, row_body, 0, unroll=8)

    for k in range(NBUF - 1):
        gather_start(k, k)

    gather_wait(0, 0)
    gather_start(NBUF - 1, NBUF - 1)
    add_pos(0)
    store_start(0, 0)

    def full_step(c, b):
        gather_wait(c, b)
        nxt = (b + NBUF - 1) % NBUF
        store_wait(c - 1, nxt)
        gather_start(c + NBUF - 1, nxt)
        add_pos(b)
        store_start(c, b)

    def group_body(i, carry):
        for k in range(NBUF):
            c = NBUF * i + 1 + k
            full_step(c, (1 + k) % NBUF)
        return carry

    n_full = SEQ_PER_WORKER - NBUF
    lax.fori_loop(0, n_full // NBUF, group_body, 0)

    for c in range(SEQ_PER_WORKER - NBUF + 1, SEQ_PER_WORKER):
        b = c % NBUF
        gather_wait(c, b)
        add_pos(b)
        store_start(c, b)

    for c in range(SEQ_PER_WORKER - NBUF, SEQ_PER_WORKER):
        store_wait(c, c % NBUF)


def kernel(input_ids, word_table, pos_table):
    ids = input_ids.astype(jnp.int32)
    out = _embed(ids, word_table, pos_table)
    return out.reshape(BATCH, SEQ_LEN, EMBED_DIM)


# speedup vs baseline: 1.1931x; 1.1931x over previous
import functools

import jax
import jax.numpy as jnp
from jax import lax
from jax.experimental import pallas as pl
from jax.experimental.pallas import tpu as pltpu
from jax.experimental.pallas import tpu_sc as plsc

VOCAB = 1000000
EMBED_DIM = 64
SEQ_LEN = 200
BATCH = 4096

NUM_CORES = 2
NUM_SUBCORES = 16
NUM_WORKERS = NUM_CORES * NUM_SUBCORES
BLK = BATCH // NUM_WORKERS
NBUF = 4

_mesh = plsc.VectorSubcoreMesh(core_axis_name="c", subcore_axis_name="s")


@functools.partial(
    pl.kernel,
    mesh=_mesh,
    out_type=jax.ShapeDtypeStruct((SEQ_LEN, BATCH, EMBED_DIM), jnp.float32),
    scratch_types=[
        pltpu.VMEM((SEQ_LEN, BLK), jnp.int32),
        [pltpu.VMEM((BLK, EMBED_DIM), jnp.float32) for _ in range(NBUF)],
        [pltpu.SemaphoreType.DMA for _ in range(NBUF)],
        [pltpu.SemaphoreType.DMA for _ in range(NBUF)],
    ],
    compiler_params=pltpu.CompilerParams(use_tc_tiling_on_sc=False),
)
def _embed(ids_t_hbm, wt_hbm, pos_rep_hbm, out_hbm, idx_all, rows, gsem, ssem):
    wid = lax.axis_index("s") * NUM_CORES + lax.axis_index("c")
    pltpu.sync_copy(ids_t_hbm.at[:, pl.ds(wid * BLK, BLK)], idx_all)

    def prep_start(s, b):
        pltpu.sync_copy(pos_rep_hbm.at[s], rows[b])
        pltpu.async_copy(wt_hbm.at[idx_all.at[s]], rows[b], gsem[b], add=True)

    def gather_wait(s, b):
        pltpu.make_async_copy(wt_hbm.at[idx_all.at[s]], rows[b], gsem[b]).wait()

    def out_slice(s):
        return out_hbm.at[s, pl.ds(wid * BLK, BLK)]

    def store_start(s, b):
        pltpu.async_copy(rows[b], out_slice(s), ssem[b])

    def store_wait(s, b):
        pltpu.make_async_copy(rows[b], out_slice(s), ssem[b]).wait()

    for k in range(NBUF - 1):
        prep_start(k, k)

    for s in range(NBUF):
        b = s % NBUF
        gather_wait(s, b)
        store_start(s, b)
        nb = (b + NBUF - 1) % NBUF
        if s > 0:
            store_wait(s - 1, nb)
        prep_start(s + NBUF - 1, nb)

    def full_step(s, b):
        gather_wait(s, b)
        store_start(s, b)
        nb = (b + NBUF - 1) % NBUF
        store_wait(s - 1, nb)
        prep_start(s + NBUF - 1, nb)

    def group_body(i, carry):
        for k in range(NBUF):
            s = NBUF * (i + 1) + k
            full_step(s, k)
        return carry

    n_full = SEQ_LEN - 2 * NBUF
    lax.fori_loop(0, n_full // NBUF, group_body, 0)

    s_last = SEQ_LEN - NBUF
    full_step(s_last, s_last % NBUF)

    for s in range(SEQ_LEN - NBUF + 1, SEQ_LEN):
        b = s % NBUF
        gather_wait(s, b)
        store_start(s, b)
        store_wait(s - 1, (b + NBUF - 1) % NBUF)

    store_wait(SEQ_LEN - 1, (SEQ_LEN - 1) % NBUF)


def kernel(input_ids, word_table, pos_table):
    ids_t = input_ids.T.astype(jnp.int32)
    pos_rep = jnp.broadcast_to(pos_table[:, None, :], (SEQ_LEN, BLK, EMBED_DIM))
    out_t = _embed(ids_t, word_table, pos_rep)
    return out_t.transpose(1, 0, 2)
